# baseline (speedup 1.0000x reference)
import functools

import jax
import jax.numpy as jnp
from jax import lax
from jax.experimental import pallas as pl
from jax.experimental.pallas import tpu as pltpu
from jax.experimental.pallas import tpu_sc as plsc

B, N, C, K = 8, 2048, 7, 16
M = B * N
FEAT = 512
BI = 256
GB = 2
MG = GB * N
RB = 2048

_HIGH = jax.lax.Precision.HIGHEST


def _knn_body(colb_ref, rowb_ref, w1aT_ref, b1a_ref, w1bT_ref, b1b_ref,
              w1cT_ref, b1c_ref, idx_ref, x64_ref):
    b = pl.program_id(0)
    cols = colb_ref[0]
    rows = rowb_ref[0]
    acc = jnp.dot(rows, cols, precision=_HIGH)
    xx_i = jnp.sum(rows * rows, axis=1, keepdims=True)
    xx_j = jnp.sum(cols * cols, axis=0, keepdims=True)
    d = 2.0 * acc - xx_i - xx_j
    niota = -lax.broadcasted_iota(jnp.int32, (BI, N), 1).astype(jnp.float32)

    idx_cols = []
    oh0 = oh1 = None
    for k in range(K):
        m = jnp.max(d, axis=1, keepdims=True)
        eq = d == m
        cand = jnp.where(eq, niota, -float(2 * N))
        jneg = jnp.max(cand, axis=1, keepdims=True)
        if k == 0:
            oh0 = (cand == jneg).astype(jnp.float32)
        elif k == 1:
            oh1 = (cand == jneg).astype(jnp.float32)
        idx_cols.append((-jneg).astype(jnp.int32))
        d = jnp.where(eq, -jnp.inf, d)
    idx_ref[0] = jnp.concatenate(idx_cols, axis=1) + b * N

    dnums = (((1,), (1,)), ((), ()))
    x0 = lax.dot_general(oh0, cols, dnums, precision=_HIGH)
    x1 = lax.dot_general(oh1, cols, dnums, precision=_HIGH)
    covs = [rows[:, :C]]
    for i in range(C):
        for j in range(C):
            covs.append(x0[:, i:i + 1] * x1[:, j:j + 1])
    feats = jnp.concatenate(covs, axis=1)

    h = jnp.maximum(jnp.dot(feats, w1aT_ref[...], precision=_HIGH) + b1a_ref[...], 0.0)
    h = jnp.maximum(jnp.dot(h, w1bT_ref[...], precision=_HIGH) + b1b_ref[...], 0.0)
    h = jnp.maximum(jnp.dot(h, w1cT_ref[...], precision=_HIGH) + b1c_ref[...], 0.0)
    x64_ref[0] = jnp.concatenate([h, jnp.zeros((BI, 64), jnp.float32)], axis=1)


def _knn_call(ptsB, ptsA, w1aT, b1a, w1bT, b1b, w1cT, b1c):
    wspec = lambda shp: pl.BlockSpec(shp, lambda b, i: (0,) * len(shp))
    return pl.pallas_call(
        _knn_body,
        grid=(GB, N // BI),
        in_specs=[
            pl.BlockSpec((1, 8, N), lambda b, i: (b, 0, 0)),
            pl.BlockSpec((1, BI, 8), lambda b, i: (b, i, 0)),
            wspec((56, 64)), wspec((1, 64)),
            wspec((64, 64)), wspec((1, 64)),
            wspec((64, 64)), wspec((1, 64)),
        ],
        out_specs=[
            pl.BlockSpec((1, BI, K), lambda b, i: (b, i, 0)),
            pl.BlockSpec((1, BI, 128), lambda b, i: (b, i, 0)),
        ],
        out_shape=[
            jax.ShapeDtypeStruct((GB, N, K), jnp.int32),
            jax.ShapeDtypeStruct((GB, N, 128), jnp.float32),
        ],
        compiler_params=pltpu.CompilerParams(
            dimension_semantics=("parallel", "parallel")),
    )(ptsB, ptsA, w1aT, b1a, w1bT, b1b, w1cT, b1c)


@functools.lru_cache(maxsize=None)
def _make_pool(D_tab, D_out, P):
    info = plsc.get_sparse_core_info()
    nw = info.num_cores * info.num_subcores
    m_per_w = MG // nw

    NB = 4
    nch = m_per_w // P
    assert nch % NB == 0

    @functools.partial(
        pl.kernel,
        out_type=jax.ShapeDtypeStruct((MG, D_out), jnp.float32),
        mesh=plsc.VectorSubcoreMesh(core_axis_name="c", subcore_axis_name="s"),
        scratch_types=[
            pltpu.VMEM((m_per_w * K,), jnp.int32),
            [pltpu.VMEM((P * K, D_tab), jnp.float32) for _ in range(NB)],
            pltpu.VMEM((P, D_out), jnp.float32),
            [pltpu.SemaphoreType.DMA for _ in range(NB)],
        ],
    )
    def pool(table_hbm, idx_hbm, out_hbm, idx_all, rows, out_v, sems):
        wid = lax.axis_index("s") * info.num_cores + lax.axis_index("c")
        base = wid * m_per_w

        pltpu.sync_copy(idx_hbm.at[pl.ds(base * K, m_per_w * K)], idx_all)

        def gather(t, buf):
            pltpu.async_copy(
                table_hbm.at[idx_all.at[pl.ds(t * (P * K), P * K)]],
                rows[buf], sems[buf])

        for buf in range(NB):
            gather(buf, buf)

        def rnd(i, carry):
            for buf in range(NB):
                t = NB * i + buf
                pltpu.make_async_copy(
                    table_hbm.at[idx_all.at[pl.ds(0, P * K)]],
                    rows[buf], sems[buf]).wait()
                rv = rows[buf]

                def point(pt, c2):
                    for cg in range(D_out // 16):
                        acc = rv[pt * K, pl.ds(cg * 16, 16)]
                        for kk in range(1, K):
                            acc = jnp.maximum(
                                acc, rv[pt * K + kk, pl.ds(cg * 16, 16)])
                        out_v[pt, pl.ds(cg * 16, 16)] = acc
                    return c2

                lax.fori_loop(0, P, point, 0)
                pltpu.sync_copy(out_v, out_hbm.at[pl.ds(base + t * P, P)])

                @pl.when(t + NB < nch)
                def _():
                    gather(t + NB, buf)
            return carry

        lax.fori_loop(0, nch // NB, rnd, 0)

    return pool


def _mid_body(x_ref, wl1T_ref, bl1_ref, wc1T_ref, bc1_ref, o_ref):
    y = jnp.dot(x_ref[...], wl1T_ref[...], precision=_HIGH) + bl1_ref[...]
    o_ref[...] = jnp.maximum(
        jnp.dot(y, wc1T_ref[...], precision=_HIGH) + bc1_ref[...], 0.0)


def _tail_body(x_ref, wl2T_ref, bl2_ref, wc2T_ref, bc2_ref, o_ref):
    y = jnp.dot(x_ref[...], wl2T_ref[...], precision=_HIGH) + bl2_ref[...]
    o_ref[...] = jnp.dot(y, wc2T_ref[...], precision=_HIGH) + bc2_ref[...]


def _dense_call(body, x, w1, bias1, w2, bias2, dout):
    din = x.shape[1]
    wspec = lambda shp: pl.BlockSpec(shp, lambda i: (0,) * len(shp))
    return pl.pallas_call(
        body,
        grid=(MG // RB,),
        in_specs=[
            pl.BlockSpec((RB, din), lambda i: (i, 0)),
            wspec(w1.shape), wspec((1, bias1.shape[-1])),
            wspec(w2.shape), wspec((1, bias2.shape[-1])),
        ],
        out_specs=pl.BlockSpec((RB, dout), lambda i: (i, 0)),
        out_shape=jax.ShapeDtypeStruct((MG, dout), jnp.float32),
        compiler_params=pltpu.CompilerParams(
            dimension_semantics=("parallel",)),
    )(x, w1, bias1.reshape(1, -1), w2, bias2.reshape(1, -1))


def _run_group(ptsg, W1a, b1a, W1b, b1b, W1c, b1c, Wl1, bl1, Wc1, bc1,
               Wl2, bl2, Wc2, bc2):
    ptsA = jnp.concatenate(
        [ptsg, jnp.zeros((GB, N, 1), jnp.float32)], axis=2)
    ptsB = jnp.swapaxes(ptsA, 1, 2)
    idx, x64 = _knn_call(
        ptsB, ptsA,
        W1a.T, b1a.reshape(1, -1),
        W1b.T, b1b.reshape(1, -1),
        W1c.T, b1c.reshape(1, -1))
    idx_flat = idx.reshape(MG * K)
    p1 = _make_pool(128, 64, 8)(x64.reshape(MG, 128), idx_flat)
    x128 = _dense_call(_mid_body, p1, Wl1.T, bl1, Wc1.T, bc1, 128)
    p2 = _make_pool(128, 128, 8)(x128, idx_flat)
    feat = _dense_call(_tail_body, p2, Wl2.T, bl2, Wc2.T, bc2, FEAT)
    return feat.reshape(GB, N, FEAT)


@jax.jit
def kernel(pts, W1a, b1a, W1b, b1b, W1c, b1c, Wl1, bl1, Wc1, bc1,
           Wl2, bl2, Wc2, bc2):
    args = (W1a, b1a, W1b, b1b, W1c, b1c, Wl1, bl1, Wc1, bc1,
            Wl2, bl2, Wc2, bc2)
    outs = [_run_group(pts[g * GB:(g + 1) * GB], *args)
            for g in range(B // GB)]
    return jnp.concatenate(outs, axis=0)


# speedup vs baseline: 1.0281x; 1.0281x over previous
import functools

import jax
import jax.numpy as jnp
from jax import lax
from jax.experimental import pallas as pl
from jax.experimental.pallas import tpu as pltpu
from jax.experimental.pallas import tpu_sc as plsc

B, N, C, K = 8, 2048, 7, 16
M = B * N
FEAT = 512
BI = 256
GB = 4
MG = GB * N
RB = 2048

_HIGH = jax.lax.Precision.HIGHEST


def _knn_body(colb_ref, rowb_ref, niota_ref, w1aT_ref, b1a_ref, w1bT_ref,
              b1b_ref, w1cT_ref, b1c_ref, idx_ref, x64_ref):
    b = pl.program_id(0)
    cols = colb_ref[0]
    rows = rowb_ref[0]
    acc = jnp.dot(rows, cols, precision=_HIGH)
    xx_i = jnp.sum(rows * rows, axis=1, keepdims=True)
    xx_j = jnp.sum(cols * cols, axis=0, keepdims=True)
    d = 2.0 * acc - xx_i - xx_j
    niota = niota_ref[...]

    idx_cols = []
    oh0 = oh1 = None
    for k in range(K):
        m = jnp.max(d, axis=1, keepdims=True)
        eq = d == m
        cand = jnp.where(eq, niota, -float(2 * N))
        jneg = jnp.max(cand, axis=1, keepdims=True)
        if k == 0:
            oh0 = (cand == jneg).astype(jnp.float32)
        elif k == 1:
            oh1 = (cand == jneg).astype(jnp.float32)
        idx_cols.append((-jneg).astype(jnp.int32))
        d = jnp.where(eq, -jnp.inf, d)
    idx_ref[0] = jnp.concatenate(idx_cols, axis=1) + b * N

    dnums = (((1,), (1,)), ((), ()))
    x0 = lax.dot_general(oh0, cols, dnums, precision=_HIGH)
    x1 = lax.dot_general(oh1, cols, dnums, precision=_HIGH)
    covs = [rows[:, :C]]
    for i in range(C):
        for j in range(C):
            covs.append(x0[:, i:i + 1] * x1[:, j:j + 1])
    feats = jnp.concatenate(covs, axis=1)

    h = jnp.maximum(jnp.dot(feats, w1aT_ref[...], precision=_HIGH) + b1a_ref[...], 0.0)
    h = jnp.maximum(jnp.dot(h, w1bT_ref[...], precision=_HIGH) + b1b_ref[...], 0.0)
    h = jnp.maximum(jnp.dot(h, w1cT_ref[...], precision=_HIGH) + b1c_ref[...], 0.0)
    x64_ref[0] = jnp.concatenate([h, jnp.zeros((BI, 64), jnp.float32)], axis=1)


def _knn_call(ptsB, ptsA, niota, w1aT, b1a, w1bT, b1b, w1cT, b1c):
    wspec = lambda shp: pl.BlockSpec(shp, lambda b, i: (0,) * len(shp))
    return pl.pallas_call(
        _knn_body,
        grid=(GB, N // BI),
        in_specs=[
            pl.BlockSpec((1, 8, N), lambda b, i: (b, 0, 0)),
            pl.BlockSpec((1, BI, 8), lambda b, i: (b, i, 0)),
            wspec((1, N)),
            wspec((56, 64)), wspec((1, 64)),
            wspec((64, 64)), wspec((1, 64)),
            wspec((64, 64)), wspec((1, 64)),
        ],
        out_specs=[
            pl.BlockSpec((1, BI, K), lambda b, i: (b, i, 0)),
            pl.BlockSpec((1, BI, 128), lambda b, i: (b, i, 0)),
        ],
        out_shape=[
            jax.ShapeDtypeStruct((GB, N, K), jnp.int32),
            jax.ShapeDtypeStruct((GB, N, 128), jnp.float32),
        ],
        compiler_params=pltpu.CompilerParams(
            dimension_semantics=("parallel", "parallel")),
    )(ptsB, ptsA, niota, w1aT, b1a, w1bT, b1b, w1cT, b1c)


@functools.lru_cache(maxsize=None)
def _make_pool(D_tab, D_out, P):
    info = plsc.get_sparse_core_info()
    nw = info.num_cores * info.num_subcores
    m_per_w = MG // nw

    NB = 4
    nch = m_per_w // P
    assert nch % NB == 0

    @functools.partial(
        pl.kernel,
        out_type=jax.ShapeDtypeStruct((MG, D_out), jnp.float32),
        mesh=plsc.VectorSubcoreMesh(core_axis_name="c", subcore_axis_name="s"),
        scratch_types=[
            pltpu.VMEM((m_per_w * K,), jnp.int32),
            [pltpu.VMEM((P * K, D_tab), jnp.float32) for _ in range(NB)],
            pltpu.VMEM((P, D_out), jnp.float32),
            [pltpu.SemaphoreType.DMA for _ in range(NB)],
        ],
    )
    def pool(table_hbm, idx_hbm, out_hbm, idx_all, rows, out_v, sems):
        wid = lax.axis_index("s") * info.num_cores + lax.axis_index("c")
        base = wid * m_per_w

        pltpu.sync_copy(idx_hbm.at[pl.ds(base * K, m_per_w * K)], idx_all)

        def gather(t, buf):
            pltpu.async_copy(
                table_hbm.at[idx_all.at[pl.ds(t * (P * K), P * K)]],
                rows[buf], sems[buf])

        for buf in range(NB):
            gather(buf, buf)

        def rnd(i, carry):
            for buf in range(NB):
                t = NB * i + buf
                pltpu.make_async_copy(
                    table_hbm.at[idx_all.at[pl.ds(0, P * K)]],
                    rows[buf], sems[buf]).wait()
                rv = rows[buf]

                def point(pt, c2):
                    for cg in range(D_out // 16):
                        acc = rv[pt * K, pl.ds(cg * 16, 16)]
                        for kk in range(1, K):
                            acc = jnp.maximum(
                                acc, rv[pt * K + kk, pl.ds(cg * 16, 16)])
                        out_v[pt, pl.ds(cg * 16, 16)] = acc
                    return c2

                lax.fori_loop(0, P, point, 0)
                pltpu.sync_copy(out_v, out_hbm.at[pl.ds(base + t * P, P)])

                @pl.when(t + NB < nch)
                def _():
                    gather(t + NB, buf)
            return carry

        lax.fori_loop(0, nch // NB, rnd, 0)

    return pool


def _mid_body(x_ref, wl1T_ref, bl1_ref, wc1T_ref, bc1_ref, o_ref):
    y = jnp.dot(x_ref[...], wl1T_ref[...], precision=_HIGH) + bl1_ref[...]
    o_ref[...] = jnp.maximum(
        jnp.dot(y, wc1T_ref[...], precision=_HIGH) + bc1_ref[...], 0.0)


def _tail_body(x_ref, wl2T_ref, bl2_ref, wc2T_ref, bc2_ref, o_ref):
    y = jnp.dot(x_ref[...], wl2T_ref[...], precision=_HIGH) + bl2_ref[...]
    o_ref[...] = jnp.dot(y, wc2T_ref[...], precision=_HIGH) + bc2_ref[...]


def _dense_call(body, x, w1, bias1, w2, bias2, dout):
    din = x.shape[1]
    wspec = lambda shp: pl.BlockSpec(shp, lambda i: (0,) * len(shp))
    return pl.pallas_call(
        body,
        grid=(MG // RB,),
        in_specs=[
            pl.BlockSpec((RB, din), lambda i: (i, 0)),
            wspec(w1.shape), wspec((1, bias1.shape[-1])),
            wspec(w2.shape), wspec((1, bias2.shape[-1])),
        ],
        out_specs=pl.BlockSpec((RB, dout), lambda i: (i, 0)),
        out_shape=jax.ShapeDtypeStruct((MG, dout), jnp.float32),
        compiler_params=pltpu.CompilerParams(
            dimension_semantics=("parallel",)),
    )(x, w1, bias1.reshape(1, -1), w2, bias2.reshape(1, -1))


def _run_group(ptsg, W1a, b1a, W1b, b1b, W1c, b1c, Wl1, bl1, Wc1, bc1,
               Wl2, bl2, Wc2, bc2):
    ptsA = jnp.concatenate(
        [ptsg, jnp.zeros((GB, N, 1), jnp.float32)], axis=2)
    ptsB = jnp.swapaxes(ptsA, 1, 2)
    niota = -jnp.arange(N, dtype=jnp.float32).reshape(1, N)
    idx, x64 = _knn_call(
        ptsB, ptsA, niota,
        W1a.T, b1a.reshape(1, -1),
        W1b.T, b1b.reshape(1, -1),
        W1c.T, b1c.reshape(1, -1))
    idx_flat = idx.reshape(MG * K)
    p1 = _make_pool(128, 64, 8)(x64.reshape(MG, 128), idx_flat)
    x128 = _dense_call(_mid_body, p1, Wl1.T, bl1, Wc1.T, bc1, 128)
    p2 = _make_pool(128, 128, 8)(x128, idx_flat)
    feat = _dense_call(_tail_body, p2, Wl2.T, bl2, Wc2.T, bc2, FEAT)
    return feat.reshape(GB, N, FEAT)


@jax.jit
def kernel(pts, W1a, b1a, W1b, b1b, W1c, b1c, Wl1, bl1, Wc1, bc1,
           Wl2, bl2, Wc2, bc2):
    args = (W1a, b1a, W1b, b1b, W1c, b1c, Wl1, bl1, Wc1, bc1,
            Wl2, bl2, Wc2, bc2)
    outs = [_run_group(pts[g * GB:(g + 1) * GB], *args)
            for g in range(B // GB)]
    return jnp.concatenate(outs, axis=0)
